# Initial kernel scaffold; baseline (speedup 1.0000x reference)
import jax
import jax.numpy as jnp
from jax.experimental import pallas as pl


def kernel(value_tok0, value_tok2, value_mask0, value_mask2, edges, orders, value_edge_ids, proj_tok_table, node_table_padded, edge_table_padded):
    raise NotImplementedError("write your pallas kernel here")


# speedup vs baseline: 1.1658x; 1.1658x over previous
import jax
import jax.numpy as jnp
from jax import lax
from jax.experimental import pallas as pl
from jax.experimental.pallas import tpu as pltpu


def _cdiv(a, b):
    return (a + b - 1) // b


def _edge_slot_kernel(tok0_ref, tok2_ref, side_ref, ptab_ref, ntab_ref,
                      etab_ref, out_ref):
    TE, S = tok0_ref.shape
    Vp, D = ptab_ref.shape
    NNp = ntab_ref.shape[0]
    NEp = etab_ref.shape[0]

    side = side_ref[...]
    len0 = side[:, 5:6]
    len2 = side[:, 6:7]

    iota_s = lax.broadcasted_iota(jnp.int32, (TE, S), 1)
    mid0 = jnp.where(iota_s < len0, tok0_ref[...], -1)
    mid2 = jnp.where(iota_s < len2, tok2_ref[...], -1)
    mids = jnp.concatenate([mid0, mid2], axis=0)

    iota_v = lax.broadcasted_iota(jnp.int32, (2 * TE, Vp), 1)
    ptab = ptab_ref[...]
    accv = jnp.zeros((2 * TE, D), jnp.float32)
    for s in range(S):
        oh = jnp.where(mids[:, s:s + 1] == iota_v, 1.0, 0.0)
        accv = accv + jnp.dot(oh, ptab, preferred_element_type=jnp.float32)

    inv0 = 1.0 / jnp.maximum(len0.astype(jnp.float32), 1.0)
    inv2 = 1.0 / jnp.maximum(len2.astype(jnp.float32), 1.0)
    val0 = accv[:TE] * inv0
    val2 = accv[TE:] * inv2

    nid = jnp.concatenate([side[:, 0:1], side[:, 1:2]], axis=0)
    iota_n = lax.broadcasted_iota(jnp.int32, (2 * TE, NNp), 1)
    ohn = jnp.where(nid == iota_n, 1.0, 0.0)
    accn = jnp.dot(ohn, ntab_ref[...], preferred_element_type=jnp.float32)

    iota_e = lax.broadcasted_iota(jnp.int32, (TE, NEp), 1)
    eh = (jnp.where(side[:, 2:3] == iota_e, 1.0, 0.0)
          + jnp.where(side[:, 3:4] == iota_e, 1.0, 0.0)
          + jnp.where(side[:, 4:5] == iota_e, 1.0, 0.0))
    edge_sum = jnp.dot(eh, etab_ref[...], preferred_element_type=jnp.float32)

    third = jnp.float32(1.0 / 3.0)
    out_ref[0] = (accn[:TE] + val0) * third
    out_ref[1] = edge_sum * third
    out_ref[2] = (accn[TE:] + val2) * third


def _run_edge_slots(tok0, tok2, side, ptab, ntab, etab, *, tile_e=512):
    E, S = tok0.shape
    Vp, D = ptab.shape
    SC = side.shape[1]

    TE = min(tile_e, _cdiv(E, 8) * 8)
    E_pad = _cdiv(E, TE) * TE

    def pad_e(x):
        if E_pad == E:
            return x
        return jnp.pad(x, [(0, E_pad - E)] + [(0, 0)] * (x.ndim - 1))

    tok0, tok2, side = pad_e(tok0), pad_e(tok2), pad_e(side)

    grid = (E_pad // TE,)
    in_specs = [
        pl.BlockSpec((TE, S), lambda i: (i, 0)),
        pl.BlockSpec((TE, S), lambda i: (i, 0)),
        pl.BlockSpec((TE, SC), lambda i: (i, 0)),
        pl.BlockSpec((Vp, D), lambda i: (0, 0)),
        pl.BlockSpec((ntab.shape[0], D), lambda i: (0, 0)),
        pl.BlockSpec((etab.shape[0], D), lambda i: (0, 0)),
    ]
    out_specs = pl.BlockSpec((3, TE, D), lambda i: (0, i, 0))

    flops = 2 * E_pad * D * (2 * S * Vp + 2 * ntab.shape[0] + etab.shape[0])
    bytes_accessed = ((tok0.size + tok2.size + side.size) * 4
                      + (Vp + ntab.shape[0] + etab.shape[0]) * D * 4
                      + 3 * E_pad * D * 4)
    cost = pl.CostEstimate(flops=flops, transcendentals=0,
                           bytes_accessed=bytes_accessed)

    out = pl.pallas_call(
        _edge_slot_kernel,
        out_shape=jax.ShapeDtypeStruct((3, E_pad, D), jnp.float32),
        grid=grid,
        in_specs=in_specs,
        out_specs=out_specs,
        compiler_params=pltpu.CompilerParams(
            dimension_semantics=("parallel",)),
        cost_estimate=cost,
    )(tok0, tok2, side, ptab, ntab, etab)

    if E_pad != E:
        out = out[:, :E, :]
    return out


def kernel(value_tok0, value_tok2, value_mask0, value_mask2,
           edges, orders, value_edge_ids,
           proj_tok_table, node_table_padded, edge_table_padded):
    E = edges.shape[0]
    len0 = jnp.sum(value_mask0.astype(jnp.int32), axis=1)
    len2 = jnp.sum(value_mask2.astype(jnp.int32), axis=1)
    side = jnp.stack(
        [edges[:, 0], edges[:, 2], edges[:, 1], orders[:, 1],
         value_edge_ids, len0, len2, jnp.zeros((E,), jnp.int32)],
        axis=1).astype(jnp.int32)
    return _run_edge_slots(
        value_tok0.astype(jnp.int32), value_tok2.astype(jnp.int32), side,
        proj_tok_table, node_table_padded, edge_table_padded)


# speedup vs baseline: 1.4694x; 1.2604x over previous
import jax
import jax.numpy as jnp
from jax import lax
from jax.experimental import pallas as pl
from jax.experimental.pallas import tpu as pltpu

_BIAS = 16384


def _cdiv(a, b):
    return (a + b - 1) // b


def _edge_slot_kernel(p0_ref, p2_ref, side_ref, ptab_ref, ntab_ref,
                      etab_ref, out_ref):
    TH, S = p0_ref.shape
    TE = 2 * TH
    Vp, D = ptab_ref.shape
    NNp = ntab_ref.shape[0]
    NEp = etab_ref.shape[0]

    side = side_ref[...]
    len0 = side[:, 5:6]
    len2 = side[:, 6:7]

    pair = jnp.concatenate([p0_ref[...], p2_ref[...]], axis=0)

    iota32 = (lax.broadcasted_iota(jnp.int32, (TE, Vp), 1) + _BIAS) * 65537
    iota_bf = pltpu.bitcast(iota32, jnp.bfloat16)

    ptab = ptab_ref[...]
    accv = jnp.zeros((2 * TE, D), jnp.float32)
    for s in range(S):
        a32 = jnp.broadcast_to(pair[:, s:s + 1], (TE, Vp))
        a_bf = pltpu.bitcast(a32, jnp.bfloat16)
        oh = jnp.where(a_bf == iota_bf, jnp.bfloat16(1), jnp.bfloat16(0))
        accv = accv + jnp.dot(oh, ptab, preferred_element_type=jnp.float32)

    inv0 = 1.0 / jnp.maximum(len0.astype(jnp.float32), 1.0)
    inv2 = 1.0 / jnp.maximum(len2.astype(jnp.float32), 1.0)
    val0 = accv[:TE] * inv0
    val2 = accv[TE:] * inv2

    nid = jnp.concatenate([side[:, 0:1], side[:, 1:2]], axis=0)
    iota_n = lax.broadcasted_iota(jnp.int32, (2 * TE, NNp), 1)
    ohn = jnp.where(nid == iota_n, 1.0, 0.0)
    accn = jnp.dot(ohn, ntab_ref[...], preferred_element_type=jnp.float32)

    iota_e = lax.broadcasted_iota(jnp.int32, (TE, NEp), 1)
    eh = (jnp.where(side[:, 2:3] == iota_e, 1.0, 0.0)
          + jnp.where(side[:, 3:4] == iota_e, 1.0, 0.0)
          + jnp.where(side[:, 4:5] == iota_e, 1.0, 0.0))
    edge_sum = jnp.dot(eh, etab_ref[...], preferred_element_type=jnp.float32)

    third = jnp.float32(1.0 / 3.0)
    out_ref[0] = (accn[:TE] + val0) * third
    out_ref[1] = edge_sum * third
    out_ref[2] = (accn[TE:] + val2) * third


def _run_edge_slots(p0, p2, side, ptab, ntab, etab, *, tile_e=512):
    Eh, S = p0.shape
    E = side.shape[0]
    Vp, D = ptab.shape
    SC = side.shape[1]

    TE = min(tile_e, _cdiv(E, 16) * 16)
    E_pad = _cdiv(E, TE) * TE
    TH = TE // 2

    def pad_rows(x, rows):
        if x.shape[0] == rows:
            return x
        return jnp.pad(x, [(0, rows - x.shape[0])] + [(0, 0)] * (x.ndim - 1))

    p0 = pad_rows(p0, E_pad // 2)
    p2 = pad_rows(p2, E_pad // 2)
    side = pad_rows(side, E_pad)

    grid = (E_pad // TE,)
    in_specs = [
        pl.BlockSpec((TH, S), lambda i: (i, 0)),
        pl.BlockSpec((TH, S), lambda i: (i, 0)),
        pl.BlockSpec((TE, SC), lambda i: (i, 0)),
        pl.BlockSpec((Vp, D), lambda i: (0, 0)),
        pl.BlockSpec((ntab.shape[0], D), lambda i: (0, 0)),
        pl.BlockSpec((etab.shape[0], D), lambda i: (0, 0)),
    ]
    out_specs = pl.BlockSpec((3, TE, D), lambda i: (0, i, 0))

    flops = 2 * E_pad * D * (2 * S * Vp + 2 * ntab.shape[0] + etab.shape[0])
    bytes_accessed = ((p0.size + p2.size + side.size) * 4
                      + (Vp + ntab.shape[0]) * D * 2 + etab.shape[0] * D * 4
                      + 3 * E_pad * D * 4)
    cost = pl.CostEstimate(flops=flops, transcendentals=0,
                           bytes_accessed=bytes_accessed)

    out = pl.pallas_call(
        _edge_slot_kernel,
        out_shape=jax.ShapeDtypeStruct((3, E_pad, D), jnp.float32),
        grid=grid,
        in_specs=in_specs,
        out_specs=out_specs,
        compiler_params=pltpu.CompilerParams(
            dimension_semantics=("parallel",)),
        cost_estimate=cost,
    )(p0, p2, side, ptab, ntab, etab)

    if E_pad != E:
        out = out[:, :E, :]
    return out


def _pack_pairs(mid):
    E, S = mid.shape
    m = mid.reshape(E // 2, 2, S)
    return m[:, 0, :] | (m[:, 1, :] << 16)


def kernel(value_tok0, value_tok2, value_mask0, value_mask2,
           edges, orders, value_edge_ids,
           proj_tok_table, node_table_padded, edge_table_padded):
    E = edges.shape[0]
    S = value_tok0.shape[1]
    len0 = jnp.sum(value_mask0.astype(jnp.int32), axis=1)
    len2 = jnp.sum(value_mask2.astype(jnp.int32), axis=1)
    side = jnp.stack(
        [edges[:, 0], edges[:, 2], edges[:, 1], orders[:, 1],
         value_edge_ids, len0, len2, jnp.zeros((E,), jnp.int32)],
        axis=1).astype(jnp.int32)

    iota_s = jnp.arange(S, dtype=jnp.int32)[None, :]
    mid0 = jnp.where(iota_s < len0[:, None],
                     value_tok0.astype(jnp.int32) + _BIAS, 0)
    mid2 = jnp.where(iota_s < len2[:, None],
                     value_tok2.astype(jnp.int32) + _BIAS, 0)

    return _run_edge_slots(
        _pack_pairs(mid0), _pack_pairs(mid2), side,
        proj_tok_table.astype(jnp.bfloat16),
        node_table_padded, edge_table_padded)


# speedup vs baseline: 3.3885x; 2.3061x over previous
import jax
import jax.numpy as jnp
from jax import lax
from jax.experimental import pallas as pl
from jax.experimental.pallas import tpu as pltpu

_BIAS = 16384


def _cdiv(a, b):
    return (a + b - 1) // b


def _edge_slot_kernel(p0_ref, p2_ref, side_ref, ptab_ref, ntab_ref,
                      etab_ref, out_ref):
    TH, S = p0_ref.shape
    TE = 2 * TH
    Vp, D = ptab_ref.shape
    NNp = ntab_ref.shape[0]
    NEp = etab_ref.shape[0]

    side = side_ref[...]
    len0 = side[:, 5:6]
    len2 = side[:, 6:7]

    pair = jnp.concatenate([p0_ref[...], p2_ref[...]], axis=0)

    MBH = 16
    MB = 2 * MBH

    iota32 = (lax.broadcasted_iota(jnp.int32, (MBH, Vp), 1) + _BIAS) * 65537
    iota_bf = pltpu.bitcast(iota32, jnp.bfloat16)

    one = jnp.bfloat16(1)
    zero = jnp.bfloat16(0)
    ptab = ptab_ref[...]
    vals = []
    for mb in range(TE // MBH):
        rows = pair[mb * MBH:(mb + 1) * MBH, :]
        acc = jnp.zeros((MB, Vp), jnp.bfloat16)
        for s in range(S):
            a32 = jnp.broadcast_to(rows[:, s:s + 1], (MBH, Vp))
            a_bf = pltpu.bitcast(a32, jnp.bfloat16)
            acc = acc + jnp.where(a_bf == iota_bf, one, zero)
        vals.append(jnp.dot(acc, ptab, preferred_element_type=jnp.float32))
    accv = jnp.concatenate(vals, axis=0)

    inv0 = 1.0 / jnp.maximum(len0.astype(jnp.float32), 1.0)
    inv2 = 1.0 / jnp.maximum(len2.astype(jnp.float32), 1.0)
    val0 = accv[:TE] * inv0
    val2 = accv[TE:] * inv2

    nid = jnp.concatenate([side[:, 0:1], side[:, 1:2]], axis=0)
    iota_n = lax.broadcasted_iota(jnp.int32, (2 * TE, NNp), 1)
    ohn = jnp.where(nid == iota_n, 1.0, 0.0)
    accn = jnp.dot(ohn, ntab_ref[...], preferred_element_type=jnp.float32)

    iota_e = lax.broadcasted_iota(jnp.int32, (TE, NEp), 1)
    eh = (jnp.where(side[:, 2:3] == iota_e, 1.0, 0.0)
          + jnp.where(side[:, 3:4] == iota_e, 1.0, 0.0)
          + jnp.where(side[:, 4:5] == iota_e, 1.0, 0.0))
    edge_sum = jnp.dot(eh, etab_ref[...], preferred_element_type=jnp.float32)

    third = jnp.float32(1.0 / 3.0)
    out_ref[0] = (accn[:TE] + val0) * third
    out_ref[1] = edge_sum * third
    out_ref[2] = (accn[TE:] + val2) * third


def _run_edge_slots(p0, p2, side, ptab, ntab, etab, *, tile_e=256):
    Eh, S = p0.shape
    E = side.shape[0]
    Vp, D = ptab.shape
    SC = side.shape[1]

    TE = min(tile_e, _cdiv(E, 16) * 16)
    E_pad = _cdiv(E, TE) * TE
    TH = TE // 2

    def pad_rows(x, rows):
        if x.shape[0] == rows:
            return x
        return jnp.pad(x, [(0, rows - x.shape[0])] + [(0, 0)] * (x.ndim - 1))

    p0 = pad_rows(p0, E_pad // 2)
    p2 = pad_rows(p2, E_pad // 2)
    side = pad_rows(side, E_pad)

    grid = (E_pad // TE,)
    in_specs = [
        pl.BlockSpec((TH, S), lambda i: (i, 0)),
        pl.BlockSpec((TH, S), lambda i: (i, 0)),
        pl.BlockSpec((TE, SC), lambda i: (i, 0)),
        pl.BlockSpec((Vp, D), lambda i: (0, 0)),
        pl.BlockSpec((ntab.shape[0], D), lambda i: (0, 0)),
        pl.BlockSpec((etab.shape[0], D), lambda i: (0, 0)),
    ]
    out_specs = pl.BlockSpec((3, TE, D), lambda i: (0, i, 0))

    flops = 2 * E_pad * D * (2 * S * Vp + 2 * ntab.shape[0] + etab.shape[0])
    bytes_accessed = ((p0.size + p2.size + side.size) * 4
                      + (Vp + ntab.shape[0]) * D * 2 + etab.shape[0] * D * 4
                      + 3 * E_pad * D * 4)
    cost = pl.CostEstimate(flops=flops, transcendentals=0,
                           bytes_accessed=bytes_accessed)

    out = pl.pallas_call(
        _edge_slot_kernel,
        out_shape=jax.ShapeDtypeStruct((3, E_pad, D), jnp.float32),
        grid=grid,
        in_specs=in_specs,
        out_specs=out_specs,
        compiler_params=pltpu.CompilerParams(
            dimension_semantics=("parallel",)),
        cost_estimate=cost,
    )(p0, p2, side, ptab, ntab, etab)

    if E_pad != E:
        out = out[:, :E, :]
    return out


def _pack_pairs(mid):
    E, S = mid.shape
    m = mid.reshape(E // 2, 2, S)
    return m[:, 0, :] | (m[:, 1, :] << 16)


def _prep_and_run(value_tok0, value_tok2, value_mask0, value_mask2,
                  edges, orders, value_edge_ids,
                  ptab_bf, node_table_padded, edge_table_padded):
    E = edges.shape[0]
    S = value_tok0.shape[1]
    len0 = jnp.sum(value_mask0.astype(jnp.int32), axis=1)
    len2 = jnp.sum(value_mask2.astype(jnp.int32), axis=1)
    side = jnp.stack(
        [edges[:, 0], edges[:, 2], edges[:, 1], orders[:, 1],
         value_edge_ids, len0, len2, jnp.zeros((E,), jnp.int32)],
        axis=1).astype(jnp.int32)

    iota_s = jnp.arange(S, dtype=jnp.int32)[None, :]
    mid0 = jnp.where(iota_s < len0[:, None],
                     value_tok0.astype(jnp.int32) + _BIAS, 0)
    mid2 = jnp.where(iota_s < len2[:, None],
                     value_tok2.astype(jnp.int32) + _BIAS, 0)

    return _run_edge_slots(
        _pack_pairs(mid0), _pack_pairs(mid2), side,
        ptab_bf, node_table_padded, edge_table_padded)


def kernel(value_tok0, value_tok2, value_mask0, value_mask2,
           edges, orders, value_edge_ids,
           proj_tok_table, node_table_padded, edge_table_padded):
    return _prep_and_run(value_tok0, value_tok2, value_mask0,
                         value_mask2, edges, orders, value_edge_ids,
                         proj_tok_table.astype(jnp.bfloat16),
                         node_table_padded, edge_table_padded)


# speedup vs baseline: 3.4259x; 1.0110x over previous
import jax
import jax.numpy as jnp
from jax import lax
from jax.experimental import pallas as pl
from jax.experimental.pallas import tpu as pltpu

_BIAS = 16384


def _cdiv(a, b):
    return (a + b - 1) // b


def _edge_slot_kernel(p0_ref, p2_ref, side_ref, ptab_ref, ntab_ref,
                      etab_ref, out_ref):
    TH, S = p0_ref.shape
    TE = 2 * TH
    Vp, D = ptab_ref.shape
    NNp = ntab_ref.shape[0]
    NEp = etab_ref.shape[0]

    side = side_ref[...]
    len0 = side[:, 5:6]
    len2 = side[:, 6:7]

    pair = jnp.concatenate([p0_ref[...], p2_ref[...]], axis=0)

    MBH = 16
    MB = 2 * MBH

    iota32 = (lax.broadcasted_iota(jnp.int32, (MBH, Vp), 1) + _BIAS) * 65537
    iota_bf = pltpu.bitcast(iota32, jnp.bfloat16)

    one = jnp.bfloat16(1)
    zero = jnp.bfloat16(0)
    ptab = ptab_ref[...]
    vals = []
    for mb in range(TE // MBH):
        rows = pair[mb * MBH:(mb + 1) * MBH, :]
        acc = jnp.zeros((MB, Vp), jnp.bfloat16)
        for s in range(S):
            a32 = jnp.broadcast_to(rows[:, s:s + 1], (MBH, Vp))
            a_bf = pltpu.bitcast(a32, jnp.bfloat16)
            acc = acc + jnp.where(a_bf == iota_bf, one, zero)
        vals.append(jnp.dot(acc, ptab, preferred_element_type=jnp.float32))
    accv = jnp.concatenate(vals, axis=0)

    inv0 = 1.0 / jnp.maximum(len0.astype(jnp.float32), 1.0)
    inv2 = 1.0 / jnp.maximum(len2.astype(jnp.float32), 1.0)
    val0 = accv[:TE] * inv0
    val2 = accv[TE:] * inv2

    nid = jnp.concatenate([side[:, 0:1], side[:, 1:2]], axis=0)
    iota_n = lax.broadcasted_iota(jnp.int32, (2 * TE, NNp), 1)
    ohn = jnp.where(nid == iota_n, 1.0, 0.0)
    accn = jnp.dot(ohn, ntab_ref[...], preferred_element_type=jnp.float32)

    iota_e = lax.broadcasted_iota(jnp.int32, (TE, NEp), 1)
    eh = (jnp.where(side[:, 2:3] == iota_e, 1.0, 0.0)
          + jnp.where(side[:, 3:4] == iota_e, 1.0, 0.0)
          + jnp.where(side[:, 4:5] == iota_e, 1.0, 0.0))
    edge_sum = jnp.dot(eh, etab_ref[...], preferred_element_type=jnp.float32)

    third = jnp.float32(1.0 / 3.0)
    out_ref[0] = (accn[:TE] + val0) * third
    out_ref[1] = edge_sum * third
    out_ref[2] = (accn[TE:] + val2) * third


def _run_edge_slots(p0, p2, side, ptab, ntab, etab, *, tile_e=512):
    Eh, S = p0.shape
    E = side.shape[0]
    Vp, D = ptab.shape
    SC = side.shape[1]

    TE = min(tile_e, _cdiv(E, 16) * 16)
    E_pad = _cdiv(E, TE) * TE
    TH = TE // 2

    def pad_rows(x, rows):
        if x.shape[0] == rows:
            return x
        return jnp.pad(x, [(0, rows - x.shape[0])] + [(0, 0)] * (x.ndim - 1))

    p0 = pad_rows(p0, E_pad // 2)
    p2 = pad_rows(p2, E_pad // 2)
    side = pad_rows(side, E_pad)

    grid = (E_pad // TE,)
    in_specs = [
        pl.BlockSpec((TH, S), lambda i: (i, 0)),
        pl.BlockSpec((TH, S), lambda i: (i, 0)),
        pl.BlockSpec((TE, SC), lambda i: (i, 0)),
        pl.BlockSpec((Vp, D), lambda i: (0, 0)),
        pl.BlockSpec((ntab.shape[0], D), lambda i: (0, 0)),
        pl.BlockSpec((etab.shape[0], D), lambda i: (0, 0)),
    ]
    out_specs = pl.BlockSpec((3, TE, D), lambda i: (0, i, 0))

    flops = 2 * E_pad * D * (2 * S * Vp + 2 * ntab.shape[0] + etab.shape[0])
    bytes_accessed = ((p0.size + p2.size + side.size) * 4
                      + (Vp + ntab.shape[0]) * D * 2 + etab.shape[0] * D * 4
                      + 3 * E_pad * D * 4)
    cost = pl.CostEstimate(flops=flops, transcendentals=0,
                           bytes_accessed=bytes_accessed)

    out = pl.pallas_call(
        _edge_slot_kernel,
        out_shape=jax.ShapeDtypeStruct((3, E_pad, D), jnp.float32),
        grid=grid,
        in_specs=in_specs,
        out_specs=out_specs,
        compiler_params=pltpu.CompilerParams(
            dimension_semantics=("parallel",)),
        cost_estimate=cost,
    )(p0, p2, side, ptab, ntab, etab)

    if E_pad != E:
        out = out[:, :E, :]
    return out


def _pack_pairs(mid):
    E, S = mid.shape
    m = mid.reshape(E // 2, 2, S)
    return m[:, 0, :] | (m[:, 1, :] << 16)


def _prep_and_run(value_tok0, value_tok2, value_mask0, value_mask2,
                  edges, orders, value_edge_ids,
                  ptab_bf, node_table_padded, edge_table_padded):
    E = edges.shape[0]
    S = value_tok0.shape[1]
    len0 = jnp.sum(value_mask0.astype(jnp.int32), axis=1)
    len2 = jnp.sum(value_mask2.astype(jnp.int32), axis=1)
    side = jnp.stack(
        [edges[:, 0], edges[:, 2], edges[:, 1], orders[:, 1],
         value_edge_ids, len0, len2, jnp.zeros((E,), jnp.int32)],
        axis=1).astype(jnp.int32)

    iota_s = jnp.arange(S, dtype=jnp.int32)[None, :]
    mid0 = jnp.where(iota_s < len0[:, None],
                     value_tok0.astype(jnp.int32) + _BIAS, 0)
    mid2 = jnp.where(iota_s < len2[:, None],
                     value_tok2.astype(jnp.int32) + _BIAS, 0)

    return _run_edge_slots(
        _pack_pairs(mid0), _pack_pairs(mid2), side,
        ptab_bf, node_table_padded, edge_table_padded)


def kernel(value_tok0, value_tok2, value_mask0, value_mask2,
           edges, orders, value_edge_ids,
           proj_tok_table, node_table_padded, edge_table_padded):
    return _prep_and_run(value_tok0, value_tok2, value_mask0,
                         value_mask2, edges, orders, value_edge_ids,
                         proj_tok_table.astype(jnp.bfloat16),
                         node_table_padded, edge_table_padded)


# speedup vs baseline: 3.4603x; 1.0100x over previous
import jax
import jax.numpy as jnp
from jax import lax
from jax.experimental import pallas as pl
from jax.experimental.pallas import tpu as pltpu

_BIAS = 16384


def _cdiv(a, b):
    return (a + b - 1) // b


def _edge_slot_kernel(p0_ref, p2_ref, side_ref, ptab_ref, ntab_ref,
                      etab_ref, out_ref):
    TH, S = p0_ref.shape
    TE = 2 * TH
    Vp, D = ptab_ref.shape
    NNp = ntab_ref.shape[0]
    NEp = etab_ref.shape[0]

    side = side_ref[...]
    len0 = side[:, 5:6]
    len2 = side[:, 6:7]

    pair = jnp.concatenate([p0_ref[...], p2_ref[...]], axis=0)

    MBH = 8
    MB = 2 * MBH

    iota32 = (lax.broadcasted_iota(jnp.int32, (MBH, Vp), 1) + _BIAS) * 65537
    iota_bf = pltpu.bitcast(iota32, jnp.bfloat16)

    one = jnp.bfloat16(1)
    zero = jnp.bfloat16(0)
    ptab = ptab_ref[...]
    vals = []
    for mb in range(TE // MBH):
        rows = pair[mb * MBH:(mb + 1) * MBH, :]
        acc = jnp.zeros((MB, Vp), jnp.bfloat16)
        for s in range(S):
            a32 = jnp.broadcast_to(rows[:, s:s + 1], (MBH, Vp))
            a_bf = pltpu.bitcast(a32, jnp.bfloat16)
            acc = acc + jnp.where(a_bf == iota_bf, one, zero)
        vals.append(jnp.dot(acc, ptab, preferred_element_type=jnp.float32))
    accv = jnp.concatenate(vals, axis=0)

    inv0 = 1.0 / jnp.maximum(len0.astype(jnp.float32), 1.0)
    inv2 = 1.0 / jnp.maximum(len2.astype(jnp.float32), 1.0)
    val0 = accv[:TE] * inv0
    val2 = accv[TE:] * inv2

    nid = jnp.concatenate([side[:, 0:1], side[:, 1:2]], axis=0)
    iota_n = lax.broadcasted_iota(jnp.int32, (2 * TE, NNp), 1)
    ohn = jnp.where(nid == iota_n, 1.0, 0.0)
    accn = jnp.dot(ohn, ntab_ref[...], preferred_element_type=jnp.float32)

    iota_e = lax.broadcasted_iota(jnp.int32, (TE, NEp), 1)
    eh = (jnp.where(side[:, 2:3] == iota_e, 1.0, 0.0)
          + jnp.where(side[:, 3:4] == iota_e, 1.0, 0.0)
          + jnp.where(side[:, 4:5] == iota_e, 1.0, 0.0))
    edge_sum = jnp.dot(eh, etab_ref[...], preferred_element_type=jnp.float32)

    third = jnp.float32(1.0 / 3.0)
    out_ref[0] = (accn[:TE] + val0) * third
    out_ref[1] = edge_sum * third
    out_ref[2] = (accn[TE:] + val2) * third


def _run_edge_slots(p0, p2, side, ptab, ntab, etab, *, tile_e=512):
    Eh, S = p0.shape
    E = side.shape[0]
    Vp, D = ptab.shape
    SC = side.shape[1]

    TE = min(tile_e, _cdiv(E, 16) * 16)
    E_pad = _cdiv(E, TE) * TE
    TH = TE // 2

    def pad_rows(x, rows):
        if x.shape[0] == rows:
            return x
        return jnp.pad(x, [(0, rows - x.shape[0])] + [(0, 0)] * (x.ndim - 1))

    p0 = pad_rows(p0, E_pad // 2)
    p2 = pad_rows(p2, E_pad // 2)
    side = pad_rows(side, E_pad)

    grid = (E_pad // TE,)
    in_specs = [
        pl.BlockSpec((TH, S), lambda i: (i, 0)),
        pl.BlockSpec((TH, S), lambda i: (i, 0)),
        pl.BlockSpec((TE, SC), lambda i: (i, 0)),
        pl.BlockSpec((Vp, D), lambda i: (0, 0)),
        pl.BlockSpec((ntab.shape[0], D), lambda i: (0, 0)),
        pl.BlockSpec((etab.shape[0], D), lambda i: (0, 0)),
    ]
    out_specs = pl.BlockSpec((3, TE, D), lambda i: (0, i, 0))

    flops = 2 * E_pad * D * (2 * S * Vp + 2 * ntab.shape[0] + etab.shape[0])
    bytes_accessed = ((p0.size + p2.size + side.size) * 4
                      + (Vp + ntab.shape[0]) * D * 2 + etab.shape[0] * D * 4
                      + 3 * E_pad * D * 4)
    cost = pl.CostEstimate(flops=flops, transcendentals=0,
                           bytes_accessed=bytes_accessed)

    out = pl.pallas_call(
        _edge_slot_kernel,
        out_shape=jax.ShapeDtypeStruct((3, E_pad, D), jnp.float32),
        grid=grid,
        in_specs=in_specs,
        out_specs=out_specs,
        compiler_params=pltpu.CompilerParams(
            dimension_semantics=("parallel",)),
        cost_estimate=cost,
    )(p0, p2, side, ptab, ntab, etab)

    if E_pad != E:
        out = out[:, :E, :]
    return out


def _pack_pairs(mid):
    E, S = mid.shape
    m = mid.reshape(E // 2, 2, S)
    return m[:, 0, :] | (m[:, 1, :] << 16)


def _prep_and_run(value_tok0, value_tok2, value_mask0, value_mask2,
                  edges, orders, value_edge_ids,
                  ptab_bf, node_table_padded, edge_table_padded):
    E = edges.shape[0]
    S = value_tok0.shape[1]
    len0 = jnp.sum(value_mask0.astype(jnp.int32), axis=1)
    len2 = jnp.sum(value_mask2.astype(jnp.int32), axis=1)
    side = jnp.stack(
        [edges[:, 0], edges[:, 2], edges[:, 1], orders[:, 1],
         value_edge_ids, len0, len2, jnp.zeros((E,), jnp.int32)],
        axis=1).astype(jnp.int32)

    iota_s = jnp.arange(S, dtype=jnp.int32)[None, :]
    mid0 = jnp.where(iota_s < len0[:, None],
                     value_tok0.astype(jnp.int32) + _BIAS, 0)
    mid2 = jnp.where(iota_s < len2[:, None],
                     value_tok2.astype(jnp.int32) + _BIAS, 0)

    return _run_edge_slots(
        _pack_pairs(mid0), _pack_pairs(mid2), side,
        ptab_bf, node_table_padded, edge_table_padded)


def kernel(value_tok0, value_tok2, value_mask0, value_mask2,
           edges, orders, value_edge_ids,
           proj_tok_table, node_table_padded, edge_table_padded):
    return _prep_and_run(value_tok0, value_tok2, value_mask0,
                         value_mask2, edges, orders, value_edge_ids,
                         proj_tok_table.astype(jnp.bfloat16),
                         node_table_padded, edge_table_padded)
